# baseline (speedup 1.0000x reference)
import functools

import jax
import jax.numpy as jnp
import numpy as np
from jax import lax
from jax.experimental import pallas as pl
from jax.experimental.pallas import tpu as pltpu
from jax.experimental.pallas import tpu_sc as plsc

_N_FEATURES = 128
_P_SINGLE = 0.3
_P_DOUBLE = 0.15
_L = 16


def _rng_inputs(batch_size: int):
    with jax.ensure_compile_time_eval():
        key = jax.random.key(42)
        k1, k2 = jax.random.split(key)
        r = jax.random.uniform(k1, (batch_size,))
        n_to_drop = jnp.where(
            r < _P_DOUBLE, 2, jnp.where(r < _P_SINGLE + _P_DOUBLE, 1, 0)
        ).astype(jnp.int32)
        u = jax.random.uniform(k2, (batch_size, _N_FEATURES))
        return np.asarray(u), np.asarray(n_to_drop)


def _tc_body(u_ref, n_ref, x_ref, o_ref):
    u = u_ref[...]
    b, f = u.shape
    n = n_ref[...]
    ii = lax.broadcasted_iota(jnp.int32, (b, f), 1)
    big = jnp.int32(f)
    m1 = jnp.min(u, axis=1, keepdims=True)
    i1 = jnp.min(jnp.where(u == m1, ii, big), axis=1, keepdims=True)
    is1 = ii == i1
    u2 = jnp.where(is1, jnp.inf, u)
    m2 = jnp.min(u2, axis=1, keepdims=True)
    i2 = jnp.min(jnp.where(u2 == m2, ii, big), axis=1, keepdims=True)
    is2 = ii == i2
    drop = (is1 & (n >= 1)) | (is2 & (n >= 2))
    mask = jnp.where(drop, 0.0, 1.0)
    o_ref[...] = x_ref[...] * mask[:, None, :]


def _tc_kernel(x, u, n_to_drop):
    batch, seq, feat = x.shape
    n_b = np.broadcast_to(n_to_drop[:, None], (batch, feat))
    blk = 128
    grid = (batch // blk,)
    return pl.pallas_call(
        _tc_body,
        grid=grid,
        in_specs=[
            pl.BlockSpec((blk, feat), lambda i: (i, 0)),
            pl.BlockSpec((blk, feat), lambda i: (i, 0)),
            pl.BlockSpec((blk, seq, feat), lambda i: (i, 0, 0)),
        ],
        out_specs=pl.BlockSpec((blk, seq, feat), lambda i: (i, 0, 0)),
        out_shape=jax.ShapeDtypeStruct(x.shape, x.dtype),
    )(u, n_b, x)


def _mask_body(u_ref, n_ref, m_ref):
    u = u_ref[...]
    b, f = u.shape
    n = n_ref[...]
    ii = lax.broadcasted_iota(jnp.int32, (b, f), 1)
    big = jnp.int32(f)
    m1 = jnp.min(u, axis=1, keepdims=True)
    i1 = jnp.min(jnp.where(u == m1, ii, big), axis=1, keepdims=True)
    is1 = ii == i1
    u2 = jnp.where(is1, jnp.inf, u)
    m2 = jnp.min(u2, axis=1, keepdims=True)
    i2 = jnp.min(jnp.where(u2 == m2, ii, big), axis=1, keepdims=True)
    is2 = ii == i2
    drop = (is1 & (n >= 1)) | (is2 & (n >= 2))
    m_ref[...] = jnp.where(drop, 0.0, 1.0)


def _tc_mask(u, n_to_drop):
    batch, feat = u.shape
    n_b = np.broadcast_to(n_to_drop[:, None], (batch, feat))
    return pl.pallas_call(
        _mask_body,
        grid=(1,),
        in_specs=[
            pl.BlockSpec((batch, feat), lambda i: (0, 0)),
            pl.BlockSpec((batch, feat), lambda i: (0, 0)),
        ],
        out_specs=pl.BlockSpec((batch, feat), lambda i: (0, 0)),
        out_shape=jax.ShapeDtypeStruct((batch, feat), jnp.float32),
    )(u, n_b)


def _sc_apply(x, mask):
    batch, seq, feat = x.shape
    info = plsc.get_sparse_core_info()
    nw = info.num_cores * info.num_subcores
    rows_per_w = batch // nw
    nch = feat // _L
    mesh = plsc.VectorSubcoreMesh(core_axis_name="c", subcore_axis_name="s")

    @functools.partial(
        pl.kernel,
        out_type=jax.ShapeDtypeStruct((batch, seq, feat), jnp.float32),
        mesh=mesh,
        scratch_types=[
            pltpu.VMEM((rows_per_w, feat), jnp.float32),
            pltpu.VMEM((seq, feat), jnp.float32),
            pltpu.SemaphoreType.DMA,
        ],
    )
    def body(x_hbm, m_hbm, out_hbm, m_v, buf_v, sem):
        wid = lax.axis_index("s") * info.num_cores + lax.axis_index("c")
        base = wid * rows_per_w
        pltpu.sync_copy(m_hbm.at[pl.ds(base, rows_per_w)], m_v)

        def row_step(r, carry):
            b = base + r
            pltpu.async_copy(x_hbm.at[b], buf_v, sem).wait()
            masks = tuple(m_v[r, pl.ds(c * _L, _L)] for c in range(nch))

            def t_step(t, ms):
                for c in range(nch):
                    sl = pl.ds(c * _L, _L)
                    buf_v[t, sl] = buf_v[t, sl] * ms[c]
                return ms

            lax.fori_loop(0, seq, t_step, masks)
            pltpu.async_copy(buf_v, out_hbm.at[b], sem).wait()
            return carry

        lax.fori_loop(0, rows_per_w, row_step, 0)

    return body(x, mask)


def kernel(x):
    batch, seq, feat = x.shape
    u, n_to_drop = _rng_inputs(batch)
    mask = _tc_mask(u, n_to_drop)
    return _sc_apply(x, mask)


# speedup vs baseline: 1.4859x; 1.4859x over previous
import functools

import jax
import jax.numpy as jnp
import numpy as np
from jax import lax
from jax.experimental import pallas as pl
from jax.experimental.pallas import tpu as pltpu
from jax.experimental.pallas import tpu_sc as plsc

_N_FEATURES = 128
_P_SINGLE = 0.3
_P_DOUBLE = 0.15
_L = 16


def _rng_inputs(batch_size: int):
    with jax.ensure_compile_time_eval():
        key = jax.random.key(42)
        k1, k2 = jax.random.split(key)
        r = jax.random.uniform(k1, (batch_size,))
        n_to_drop = jnp.where(
            r < _P_DOUBLE, 2, jnp.where(r < _P_SINGLE + _P_DOUBLE, 1, 0)
        ).astype(jnp.int32)
        u = jax.random.uniform(k2, (batch_size, _N_FEATURES))
        return np.asarray(u), np.asarray(n_to_drop)


def _tc_body(u_ref, n_ref, x_ref, o_ref):
    u = u_ref[...]
    b, f = u.shape
    n = n_ref[...]
    ii = lax.broadcasted_iota(jnp.int32, (b, f), 1)
    big = jnp.int32(f)
    m1 = jnp.min(u, axis=1, keepdims=True)
    i1 = jnp.min(jnp.where(u == m1, ii, big), axis=1, keepdims=True)
    is1 = ii == i1
    u2 = jnp.where(is1, jnp.inf, u)
    m2 = jnp.min(u2, axis=1, keepdims=True)
    i2 = jnp.min(jnp.where(u2 == m2, ii, big), axis=1, keepdims=True)
    is2 = ii == i2
    drop = (is1 & (n >= 1)) | (is2 & (n >= 2))
    mask = jnp.where(drop, 0.0, 1.0)
    o_ref[...] = x_ref[...] * mask[:, None, :]


def _tc_kernel(x, u, n_to_drop):
    batch, seq, feat = x.shape
    n_b = np.broadcast_to(n_to_drop[:, None], (batch, feat))
    blk = 128
    grid = (batch // blk,)
    return pl.pallas_call(
        _tc_body,
        grid=grid,
        in_specs=[
            pl.BlockSpec((blk, feat), lambda i: (i, 0)),
            pl.BlockSpec((blk, feat), lambda i: (i, 0)),
            pl.BlockSpec((blk, seq, feat), lambda i: (i, 0, 0)),
        ],
        out_specs=pl.BlockSpec((blk, seq, feat), lambda i: (i, 0, 0)),
        out_shape=jax.ShapeDtypeStruct(x.shape, x.dtype),
    )(u, n_b, x)


def _mask_body(u_ref, n_ref, m_ref):
    u = u_ref[...]
    b, f = u.shape
    n = n_ref[...]
    ii = lax.broadcasted_iota(jnp.int32, (b, f), 1)
    big = jnp.int32(f)
    m1 = jnp.min(u, axis=1, keepdims=True)
    i1 = jnp.min(jnp.where(u == m1, ii, big), axis=1, keepdims=True)
    is1 = ii == i1
    u2 = jnp.where(is1, jnp.inf, u)
    m2 = jnp.min(u2, axis=1, keepdims=True)
    i2 = jnp.min(jnp.where(u2 == m2, ii, big), axis=1, keepdims=True)
    is2 = ii == i2
    drop = (is1 & (n >= 1)) | (is2 & (n >= 2))
    m_ref[...] = jnp.where(drop, 0.0, 1.0)


def _tc_mask(u, n_to_drop):
    batch, feat = u.shape
    n_b = np.broadcast_to(n_to_drop[:, None], (batch, feat))
    return pl.pallas_call(
        _mask_body,
        grid=(1,),
        in_specs=[
            pl.BlockSpec((batch, feat), lambda i: (0, 0)),
            pl.BlockSpec((batch, feat), lambda i: (0, 0)),
        ],
        out_specs=pl.BlockSpec((batch, feat), lambda i: (0, 0)),
        out_shape=jax.ShapeDtypeStruct((batch, feat), jnp.float32),
    )(u, n_b)


def _sc_apply(x, mask):
    batch, seq, feat = x.shape
    info = plsc.get_sparse_core_info()
    nw = info.num_cores * info.num_subcores
    rows_per_w = batch // nw
    nch = feat // _L
    mesh = plsc.VectorSubcoreMesh(core_axis_name="c", subcore_axis_name="s")

    nbuf = 4

    @functools.partial(
        pl.kernel,
        out_type=jax.ShapeDtypeStruct((batch, seq, feat), jnp.float32),
        mesh=mesh,
        scratch_types=[
            pltpu.VMEM((rows_per_w, feat), jnp.float32),
            pltpu.VMEM((nbuf, seq, feat), jnp.float32),
        ]
        + [pltpu.SemaphoreType.DMA] * (2 * nbuf),
    )
    def body(x_hbm, m_hbm, out_hbm, m_v, buf_v, *sems):
        sin, sout = sems[:nbuf], sems[nbuf:]
        wid = lax.axis_index("s") * info.num_cores + lax.axis_index("c")
        base = wid * rows_per_w
        pltpu.sync_copy(m_hbm.at[pl.ds(base, rows_per_w)], m_v)

        def start_in(r, s):
            pltpu.make_async_copy(x_hbm.at[base + r], buf_v.at[s], sin[s]).start()

        def wait_in(s):
            pltpu.make_async_copy(x_hbm.at[base], buf_v.at[s], sin[s]).wait()

        def start_out(r, s):
            pltpu.make_async_copy(buf_v.at[s], out_hbm.at[base + r], sout[s]).start()

        def wait_out(s):
            pltpu.make_async_copy(buf_v.at[s], out_hbm.at[base], sout[s]).wait()

        start_in(0, 0)
        start_in(1, 1)
        for r in range(rows_per_w):
            s = r % nbuf
            wait_in(s)
            masks = tuple(m_v[r, pl.ds(c * _L, _L)] for c in range(nch))

            def t_step(t, ms, s=s):
                for c in range(nch):
                    sl = pl.ds(c * _L, _L)
                    buf_v[s, t, sl] = buf_v[s, t, sl] * ms[c]
                return ms

            lax.fori_loop(0, seq, t_step, masks)
            start_out(r, s)
            nxt = r + 2
            if nxt < rows_per_w:
                ns = nxt % nbuf
                if nxt >= nbuf:
                    wait_out(ns)
                start_in(nxt, ns)
        for r in range(max(0, rows_per_w - nbuf), rows_per_w):
            wait_out(r % nbuf)

    return body(x, mask)


def kernel(x):
    batch, seq, feat = x.shape
    u, n_to_drop = _rng_inputs(batch)
    mask = _tc_mask(u, n_to_drop)
    return _sc_apply(x, mask)
